# Initial kernel scaffold; baseline (speedup 1.0000x reference)
import jax
import jax.numpy as jnp
from jax.experimental import pallas as pl


def kernel(x):
    raise NotImplementedError("write your pallas kernel here")


# speedup vs baseline: 4.7723x; 4.7723x over previous
import math

import jax
import jax.numpy as jnp
import numpy as np
from jax.experimental import pallas as pl

_D = 256
_NPER = 128
_DOUT = 384
_BLK = 1024

_E_NP = np.zeros((_NPER, 2 * _NPER), dtype=np.float32)
_E_NP[np.arange(_NPER), 2 * np.arange(_NPER)] = 1.0
_E_NP[np.arange(_NPER), 2 * np.arange(_NPER) + 1] = 1.0
_E = jnp.asarray(_E_NP)


def _body(x_ref, e_ref, o_ref):
    x = x_ref[...]
    x2 = jnp.dot(x[:, :_NPER], e_ref[...], preferred_element_type=jnp.float32)
    lane = jax.lax.broadcasted_iota(jnp.int32, (_BLK, 2 * _NPER), 1)
    phase = jnp.where(
        lane % 2 == 0, jnp.float32(math.pi), jnp.float32(math.pi / 2)
    )
    o_ref[:, : 2 * _NPER] = jnp.cos(x2 + phase)
    o_ref[:, 2 * _NPER :] = x[:, _NPER:]


def kernel(x):
    batch = x.shape[0]
    grid = (batch // _BLK,)
    return pl.pallas_call(
        _body,
        grid=grid,
        in_specs=[
            pl.BlockSpec((_BLK, _D), lambda i: (i, 0)),
            pl.BlockSpec((_NPER, 2 * _NPER), lambda i: (0, 0)),
        ],
        out_specs=pl.BlockSpec((_BLK, _DOUT), lambda i: (i, 0)),
        out_shape=jax.ShapeDtypeStruct((batch, _DOUT), x.dtype),
    )(x, _E)


# speedup vs baseline: 12.1903x; 2.5544x over previous
import math

import jax
import jax.numpy as jnp
import numpy as np
from jax.experimental import pallas as pl

_D = 256
_NPER = 128
_DOUT = 384
_BLK = 1024

_E_NP = np.zeros((_NPER, 2 * _NPER), dtype=np.float32)
_E_NP[np.arange(_NPER), 2 * np.arange(_NPER)] = 1.0
_E_NP[np.arange(_NPER), 2 * np.arange(_NPER) + 1] = 1.0
_E = jnp.asarray(_E_NP)


_C0 = 9.999994010e-01
_C1 = -4.999953021e-01
_C2 = 4.166075139e-02
_C3 = -1.386178414e-03
_C4 = 2.424003292e-05
_C5 = -2.213212478e-07
_INV_2PI = 0.15915494309189535
_TWOPI_HI = 6.28125
_TWOPI_LO = 0.0019353071795864769
def _fast_cos(t):
    n = jnp.round(t * jnp.float32(_INV_2PI))
    r = t - n * jnp.float32(_TWOPI_HI)
    r = r - n * jnp.float32(_TWOPI_LO)
    r2 = r * r
    p = jnp.float32(_C5)
    p = p * r2 + jnp.float32(_C4)
    p = p * r2 + jnp.float32(_C3)
    p = p * r2 + jnp.float32(_C2)
    p = p * r2 + jnp.float32(_C1)
    p = p * r2 + jnp.float32(_C0)
    return p


def _body(x_ref, e_ref, o_ref):
    x = x_ref[...]
    x2 = jnp.dot(x[:, :_NPER], e_ref[...], preferred_element_type=jnp.float32)
    lane = jax.lax.broadcasted_iota(jnp.int32, (_BLK, 2 * _NPER), 1)
    phase = jnp.where(
        lane % 2 == 0, jnp.float32(math.pi), jnp.float32(math.pi / 2)
    )
    o_ref[:, : 2 * _NPER] = _fast_cos(x2 + phase)
    o_ref[:, 2 * _NPER :] = x[:, _NPER:]


def kernel(x):
    batch = x.shape[0]
    grid = (batch // _BLK,)
    return pl.pallas_call(
        _body,
        grid=grid,
        in_specs=[
            pl.BlockSpec((_BLK, _D), lambda i: (i, 0)),
            pl.BlockSpec((_NPER, 2 * _NPER), lambda i: (0, 0)),
        ],
        out_specs=pl.BlockSpec((_BLK, _DOUT), lambda i: (i, 0)),
        out_shape=jax.ShapeDtypeStruct((batch, _DOUT), x.dtype),
    )(x, _E)


# speedup vs baseline: 17.7158x; 1.4533x over previous
import math

import jax
import jax.numpy as jnp
import numpy as np
from jax.experimental import pallas as pl

_D = 256
_NPER = 128
_DOUT = 384
_BLK = 4096

_E_NP = np.zeros((_NPER, 2 * _NPER), dtype=np.float32)
_E_NP[np.arange(_NPER), 2 * np.arange(_NPER)] = 1.0
_E_NP[np.arange(_NPER), 2 * np.arange(_NPER) + 1] = 1.0
_E = jnp.asarray(_E_NP)


_C0 = 9.999994010e-01
_C1 = -4.999953021e-01
_C2 = 4.166075139e-02
_C3 = -1.386178414e-03
_C4 = 2.424003292e-05
_C5 = -2.213212478e-07
_INV_2PI = 0.15915494309189535
_TWOPI_HI = 6.28125
_TWOPI_LO = 0.0019353071795864769
def _fast_cos(t):
    n = jnp.round(t * jnp.float32(_INV_2PI))
    r = t - n * jnp.float32(_TWOPI_HI)
    r = r - n * jnp.float32(_TWOPI_LO)
    r2 = r * r
    p = jnp.float32(_C5)
    p = p * r2 + jnp.float32(_C4)
    p = p * r2 + jnp.float32(_C3)
    p = p * r2 + jnp.float32(_C2)
    p = p * r2 + jnp.float32(_C1)
    p = p * r2 + jnp.float32(_C0)
    return p


def _body(x_ref, e_ref, o_ref):
    x = x_ref[...]
    x2 = jnp.dot(x[:, :_NPER], e_ref[...], preferred_element_type=jnp.float32)
    lane = jax.lax.broadcasted_iota(jnp.int32, (_BLK, 2 * _NPER), 1)
    phase = jnp.where(
        lane % 2 == 0, jnp.float32(math.pi), jnp.float32(math.pi / 2)
    )
    o_ref[:, : 2 * _NPER] = _fast_cos(x2 + phase)
    o_ref[:, 2 * _NPER :] = x[:, _NPER:]


def kernel(x):
    batch = x.shape[0]
    grid = (batch // _BLK,)
    return pl.pallas_call(
        _body,
        grid=grid,
        in_specs=[
            pl.BlockSpec((_BLK, _D), lambda i: (i, 0)),
            pl.BlockSpec((_NPER, 2 * _NPER), lambda i: (0, 0)),
        ],
        out_specs=pl.BlockSpec((_BLK, _DOUT), lambda i: (i, 0)),
        out_shape=jax.ShapeDtypeStruct((batch, _DOUT), x.dtype),
    )(x, _E)
